# baseline (speedup 1.0000x reference)
import functools

import jax
import jax.numpy as jnp
from jax import lax
from jax.experimental import pallas as pl
from jax.experimental.pallas import tpu as pltpu
from jax.experimental.pallas import tpu_sc as plsc

L = 16
WB = 1792
KB = 128


@functools.lru_cache(maxsize=None)
def _make_update(M, D, B, num_cores=2, num_subcores=16):
  NW = num_cores * num_subcores
  assert NW == 32 and B % L == 0 and L < D <= 2 * L
  DHI = D - L
  NFULL = M // WB
  TAIL = M - NFULL * WB
  NBLK = NFULL + (1 if TAIL else 0)
  KMAX = (NBLK + NW - 1) // NW
  NVEC = B // L

  mesh = plsc.VectorSubcoreMesh(
      core_axis_name="c", subcore_axis_name="s",
      num_cores=num_cores, num_subcores=num_subcores)

  lanes = lambda: lax.iota(jnp.int32, L)
  sp = lambda x: jnp.full((L,), x, jnp.int32)

  def body(memt_hbm, valr_hbm, idx_hbm, out_hbm,
           idx_v, jl, jl2, buf, buft, vbuf, ibuf, sem, semv):
    wid = lax.axis_index("s") * num_cores + lax.axis_index("c")
    pltpu.sync_copy(idx_hbm, idx_v)

    def scan_body(i, cnt):
      v = idx_v[pl.ds(i * L, L)]
      m = (lax.div(v, sp(WB)) & (NW - 1)) == sp(wid)
      pos = lanes() + sp(i * L)
      offs = plsc.cumsum(m.astype(jnp.int32)) - 1
      plsc.store_scatter(jl, [sp(cnt) + offs], pos, mask=m)
      return cnt + jnp.sum(m.astype(jnp.int32))
    cnt = lax.fori_loop(0, NVEC, scan_body, jnp.int32(0))

    def apply_updates(bufref, base, width):
      def sub_scan(i, n2):
        valid = (lanes() + sp(i * L)) < sp(cnt)
        j = jl[pl.ds(i * L, L)]
        r = plsc.load_gather(idx_v, [j], mask=valid)
        rel = r - sp(base)
        m = valid & (rel >= 0) & (rel < width)
        offs = plsc.cumsum(m.astype(jnp.int32)) - 1
        plsc.store_scatter(jl2, [sp(n2) + offs], j, mask=m)
        return n2 + jnp.sum(m.astype(jnp.int32))
      nv = lax.div(cnt + (L - 1), jnp.int32(L))
      n2 = lax.fori_loop(0, nv, sub_scan, jnp.int32(0))

      def batch_body(q, _):
        bb = q * KB
        nb = jnp.minimum(jnp.int32(KB), n2 - bb)
        for i in range(KB // L):
          valid = (lanes() + sp(i * L)) < sp(nb)
          j = jl2[pl.ds(bb + i * L, L)]
          jd = jnp.where(valid, j, sp(0)) * D
          for c in range(D):
            ibuf[c, pl.ds(i * L, L)] = jd + c
        hs = [pltpu.async_copy(valr_hbm.at[ibuf.at[c]], vbuf.at[c], semv)
              for c in range(D)]
        for h in hs:
          h.wait()

        def upd_body(e, _):
          je = plsc.load_gather(jl2, [sp(bb + e)])
          re = plsc.load_gather(idx_v, [je]) - sp(base)
          c_lo = lanes()
          cur = plsc.load_gather(bufref, [c_lo, re])
          v = plsc.load_gather(vbuf, [c_lo, sp(e)])
          plsc.store_scatter(bufref, [c_lo, re], cur + v)
          mhi = c_lo < DHI
          c_hi = c_lo + L
          cur1 = plsc.load_gather(bufref, [c_hi, re], mask=mhi)
          v1 = plsc.load_gather(vbuf, [c_hi, sp(e)], mask=mhi)
          plsc.store_scatter(bufref, [c_hi, re], cur1 + v1, mask=mhi)
          return 0
        lax.fori_loop(0, nb, upd_body, 0)
        return 0
      nbat = lax.div(n2 + (KB - 1), jnp.int32(KB))
      lax.fori_loop(0, nbat, batch_body, 0)

    def block_body(k, _):
      b = wid + k * NW
      base = b * WB

      @pl.when(b < NFULL)
      def _full():
        pltpu.async_copy(memt_hbm.at[:, pl.ds(base, WB)], buf, sem).wait()
        apply_updates(buf, base, WB)
        pltpu.async_copy(buf, out_hbm.at[:, pl.ds(base, WB)], sem).wait()

      if TAIL:
        @pl.when(b == NFULL)
        def _tail():
          pltpu.async_copy(memt_hbm.at[:, pl.ds(NFULL * WB, TAIL)], buft,
                           sem).wait()
          apply_updates(buft, base, TAIL)
          pltpu.async_copy(buft, out_hbm.at[:, pl.ds(NFULL * WB, TAIL)],
                           sem).wait()
      return 0
    lax.fori_loop(0, KMAX, block_body, 0)

  return pl.kernel(
      body,
      out_type=jax.ShapeDtypeStruct((D, M), jnp.float32),
      mesh=mesh,
      scratch_types=[
          pltpu.VMEM((B,), jnp.int32),
          pltpu.VMEM((B + L,), jnp.int32),
          pltpu.VMEM((B + L,), jnp.int32),
          pltpu.VMEM((D, WB), jnp.float32),
          pltpu.VMEM((D, TAIL or L), jnp.float32),
          pltpu.VMEM((D, KB), jnp.float32),
          pltpu.VMEM((D, KB), jnp.int32),
          pltpu.SemaphoreType.DMA,
          pltpu.SemaphoreType.DMA,
      ],
      compiler_params=pltpu.CompilerParams(needs_layout_passes=False),
  )


def kernel(mem, val, idx):
  M, D = mem.shape
  B = val.shape[0]
  idx32 = idx.astype(jnp.int32)
  memt = mem.T
  valr = val.reshape(B * D)
  outt = _make_update(M, D, B)(memt, valr, idx32)
  return outt.T


# speedup vs baseline: 1.1237x; 1.1237x over previous
import functools

import jax
import jax.numpy as jnp
from jax import lax
from jax.experimental import pallas as pl
from jax.experimental.pallas import tpu as pltpu
from jax.experimental.pallas import tpu_sc as plsc

L = 16
KB = 64
WB = 2000
VR = 128


@functools.lru_cache(maxsize=None)
def _make_update(M, D, B, num_cores=2, num_subcores=16):
  NW = num_cores * num_subcores
  NB = M // WB
  assert M % WB == 0 and B % L == 0 and L < D <= 2 * L
  DHI = D - L
  NVEC = B // L

  mesh = plsc.VectorSubcoreMesh(
      core_axis_name="c", subcore_axis_name="s",
      num_cores=num_cores, num_subcores=num_subcores)

  lanes = lambda: lax.iota(jnp.int32, L)
  sp = lambda x: jnp.full((L,), x, jnp.int32)

  def body(memf_hbm, valr_hbm, idx_hbm, outf_hbm,
           idx_v, jl, bl, rbl, lmbuf, offbuf, vbuf, buf, sem, semb):
    wid = lax.axis_index("s") * num_cores + lax.axis_index("c")

    pltpu.sync_copy(idx_hbm, idx_v)

    def scan_body(i, cnt):
      v = idx_v[pl.ds(i * L, L)]
      own = lax.rem(lax.div(v, sp(WB)), sp(NW)) == sp(wid)
      pos = lanes() + sp(i * L)
      offs = plsc.cumsum(own.astype(jnp.int32)) - 1
      plsc.store_scatter(jl, [sp(cnt) + offs], pos, mask=own)
      return cnt + jnp.sum(own.astype(jnp.int32))
    cnt = lax.fori_loop(0, NVEC, scan_body, jnp.int32(0))
    nvo = lax.div(cnt + jnp.int32(L - 1), jnp.int32(L))

    def block_body(bi, _):
      b = wid + bi * NW
      m0 = b * jnp.int32(WB)
      hin = [pltpu.async_copy(memf_hbm.at[pl.ds(d * M + m0, WB)],
                              buf.at[pl.ds(d * WB, WB)], semb)
             for d in range(D)]
      for h in hin:
        h.wait()

      def bscan(i, bcnt):
        pos = lanes() + sp(i * L)
        valid = pos < sp(cnt)
        j = jl[pl.ds(i * L, L)]
        jc = jnp.where(valid, j, sp(0))
        m = plsc.load_gather(idx_v, [jc])
        inb = valid & (m >= sp(m0)) & (m < sp(m0) + sp(WB))
        offs = plsc.cumsum(inb.astype(jnp.int32)) - 1
        pk = jc * sp(2048) + (m - sp(m0))
        plsc.store_scatter(bl, [sp(bcnt) + offs], pk, mask=inb)
        return bcnt + jnp.sum(inb.astype(jnp.int32))
      bcnt = lax.fori_loop(0, nvo, bscan, jnp.int32(0))

      def chunk_body(q, _):
        base = q * KB
        for i in range(KB // L):
          pos = lanes() + sp(base + i * L)
          valid = pos < sp(bcnt)
          pk = bl[pl.ds(base + i * L, L)]
          j = lax.div(pk, sp(2048))
          lm = pk - j * sp(2048)
          lmbuf[pl.ds(i * L, L)] = lm
          s = jnp.where(valid, j, sp(0)) * sp(D)
          r0 = lax.div(s, sp(VR))
          off = s - r0 * sp(VR)
          offbuf[pl.ds(i * L, L)] = off
          r1 = r0 + (off > sp(VR - D)).astype(jnp.int32)
          ppos = (lanes() + sp(i * L)) * sp(2)
          plsc.store_scatter(rbl, [ppos], r0)
          plsc.store_scatter(rbl, [ppos + sp(1)], r1)
        pltpu.async_copy(valr_hbm.at[rbl], vbuf, sem).wait()

        mhi = lanes() < sp(DHI)

        def apply_one(u, _):
          lmv = plsc.load_gather(lmbuf, [sp(u)])
          offv = plsc.load_gather(offbuf, [sp(u)])
          clo = offv + lanes()
          rs = lax.div(clo, sp(VR))
          vlo = plsc.load_gather(vbuf, [sp(2 * u) + rs, clo - rs * sp(VR)])
          plsc.addupdate_scatter(buf, [lanes() * sp(WB) + lmv], vlo)
          chi = offv + lanes() + sp(L)
          rs2 = lax.div(chi, sp(VR))
          vhi = plsc.load_gather(
              vbuf, [sp(2 * u) + rs2, chi - rs2 * sp(VR)], mask=mhi)
          plsc.addupdate_scatter(
              buf, [(lanes() + sp(L)) * sp(WB) + lmv], vhi, mask=mhi)
          return 0
        rem = jnp.minimum(jnp.int32(KB), bcnt - base)
        lax.fori_loop(0, rem, apply_one, 0)
        return 0
      nq = lax.div(bcnt + jnp.int32(KB - 1), jnp.int32(KB))
      lax.fori_loop(0, nq, chunk_body, 0)

      hout = [pltpu.async_copy(buf.at[pl.ds(d * WB, WB)],
                               outf_hbm.at[pl.ds(d * M + m0, WB)], semb)
              for d in range(D)]
      for h in hout:
        h.wait()
      return 0

    nb_w = lax.div(jnp.int32(NB) - wid + jnp.int32(NW - 1), jnp.int32(NW))
    lax.fori_loop(0, nb_w, block_body, 0)

  return pl.kernel(
      body,
      out_type=jax.ShapeDtypeStruct((D * M,), jnp.float32),
      mesh=mesh,
      scratch_types=[
          pltpu.VMEM((B,), jnp.int32),
          pltpu.VMEM((B + L,), jnp.int32),
          pltpu.VMEM((B + L,), jnp.int32),
          pltpu.VMEM((2 * KB,), jnp.int32),
          pltpu.VMEM((KB,), jnp.int32),
          pltpu.VMEM((KB,), jnp.int32),
          pltpu.VMEM((2 * KB, VR), jnp.float32),
          pltpu.VMEM((D * WB,), jnp.float32),
          pltpu.SemaphoreType.DMA,
          pltpu.SemaphoreType.DMA,
      ],
      compiler_params=pltpu.CompilerParams(needs_layout_passes=False),
  )


def kernel(mem, val, idx):
  M, D = mem.shape
  B = val.shape[0]
  idx32 = idx.astype(jnp.int32)
  memf = mem.T.reshape(D * M)
  valr = val.reshape(B * D // VR, VR)
  outf = _make_update(M, D, B)(memf, valr, idx32)
  return outf.reshape(D, M).T
